# Initial kernel scaffold; baseline (speedup 1.0000x reference)
import jax
import jax.numpy as jnp
from jax.experimental import pallas as pl


def kernel(x, edge_index, batch, W1, b1, W2, b2, Wfc, bfc):
    raise NotImplementedError("write your pallas kernel here")


# speedup vs baseline: 21.7748x; 21.7748x over previous
import functools

import jax
import jax.numpy as jnp
from jax import lax
from jax.experimental import pallas as pl
from jax.experimental.pallas import tpu as pltpu
from jax.experimental.pallas import tpu_sc as plsc

_N = 10000
_E = 320000
_G = 128

_NCORES = 2
_NSUB = 16
_NW = _NCORES * _NSUB
_CHUNK = 128

_N_PAD = 10240
_ROWS_PER_TILE = _N_PAD // _NSUB
_EPW = ((_E + _NW * _CHUNK - 1) // (_NW * _CHUNK)) * _CHUNK
_E_PAD = _EPW * _NW
_CHUNKS_PER_W = _EPW // _CHUNK

_BLK = 1024
_NBLK = _N_PAD // _BLK


_SC_PARAMS = pltpu.CompilerParams(use_tc_tiling_on_sc=False)


def _sc_degree(dst_pad):
  mesh = plsc.VectorSubcoreMesh(core_axis_name="c", subcore_axis_name="s")

  @functools.partial(
      pl.kernel,
      out_type=jax.ShapeDtypeStruct((_NCORES * _N_PAD,), jnp.float32),
      mesh=mesh,
      compiler_params=_SC_PARAMS,
      scratch_types=[
          pltpu.VMEM((_CHUNK,), jnp.int32),
          pltpu.VMEM((_CHUNK,), jnp.float32),
          pltpu.VMEM((_ROWS_PER_TILE,), jnp.float32),
          pltpu.VMEM_SHARED((_N_PAD,), jnp.float32),
      ],
  )
  def k(dst_hbm, out_hbm, idx_v, ones_v, zbuf_v, acc_sh):
    c = lax.axis_index("c")
    s = lax.axis_index("s")
    w = c * _NSUB + s
    for j in range(_CHUNK // 16):
      ones_v[pl.ds(j * 16, 16)] = jnp.ones((16,), jnp.float32)

    def zfill(i, carry):
      zbuf_v[pl.ds(i * 16, 16)] = jnp.zeros((16,), jnp.float32)
      return carry

    lax.fori_loop(0, _ROWS_PER_TILE // 16, zfill, 0)
    pltpu.sync_copy(zbuf_v, acc_sh.at[pl.ds(s * _ROWS_PER_TILE, _ROWS_PER_TILE)])
    plsc.subcore_barrier()

    def body(i, carry):
      base = w * _EPW + i * _CHUNK
      pltpu.sync_copy(dst_hbm.at[pl.ds(base, _CHUNK)], idx_v)
      pltpu.sync_copy(ones_v, acc_sh.at[idx_v], add=True)
      return carry

    lax.fori_loop(0, _CHUNKS_PER_W, body, 0)
    plsc.subcore_barrier()
    pltpu.sync_copy(
        acc_sh.at[pl.ds(s * _ROWS_PER_TILE, _ROWS_PER_TILE)],
        out_hbm.at[pl.ds(c * _N_PAD + s * _ROWS_PER_TILE, _ROWS_PER_TILE)])

  return k(dst_pad)


def _sc_aggregate(g, src_pad, dst_pad, d):
  mesh = plsc.VectorSubcoreMesh(core_axis_name="c", subcore_axis_name="s")

  @functools.partial(
      pl.kernel,
      out_type=jax.ShapeDtypeStruct((_NCORES * _N_PAD, d), jnp.float32),
      mesh=mesh,
      compiler_params=_SC_PARAMS,
      scratch_types=[
          pltpu.VMEM((_CHUNK,), jnp.int32),
          pltpu.VMEM((_CHUNK,), jnp.int32),
          pltpu.VMEM((_CHUNK, d), jnp.float32),
          pltpu.VMEM((_ROWS_PER_TILE, d), jnp.float32),
          pltpu.VMEM_SHARED((_N_PAD, d), jnp.float32),
          pltpu.SemaphoreType.DMA,
      ],
  )
  def k(g_hbm, src_hbm, dst_hbm, out_hbm, src_v, dst_v, rows_v, zbuf_v,
        acc_sh, sem):
    c = lax.axis_index("c")
    s = lax.axis_index("s")
    w = c * _NSUB + s

    def zfill(i, carry):
      for j in range(d // 16):
        zbuf_v[i, pl.ds(j * 16, 16)] = jnp.zeros((16,), jnp.float32)
      return carry

    lax.fori_loop(0, _ROWS_PER_TILE, zfill, 0)
    pltpu.sync_copy(zbuf_v, acc_sh.at[pl.ds(s * _ROWS_PER_TILE, _ROWS_PER_TILE)])
    plsc.subcore_barrier()

    def body(i, carry):
      base = w * _EPW + i * _CHUNK
      pltpu.sync_copy(src_hbm.at[pl.ds(base, _CHUNK)], src_v)
      pltpu.sync_copy(dst_hbm.at[pl.ds(base, _CHUNK)], dst_v)
      pltpu.async_copy(g_hbm.at[src_v], rows_v, sem).wait()
      pltpu.sync_copy(rows_v, acc_sh.at[dst_v], add=True)
      return carry

    lax.fori_loop(0, _CHUNKS_PER_W, body, 0)
    plsc.subcore_barrier()
    pltpu.sync_copy(
        acc_sh.at[pl.ds(s * _ROWS_PER_TILE, _ROWS_PER_TILE)],
        out_hbm.at[pl.ds(c * _N_PAD + s * _ROWS_PER_TILE, _ROWS_PER_TILE)])

  return k(g, src_pad, dst_pad)


def _tc_layer1(pdeg_flat, x_pad, W1):
  def body(pd0_ref, pd1_ref, x_ref, w_ref, g_ref, dinv_ref):
    deg = 1.0 + pd0_ref[...] + pd1_ref[...]
    dinv = lax.rsqrt(deg)
    h = jnp.dot(x_ref[...], w_ref[...], preferred_element_type=jnp.float32)
    g_ref[...] = h * dinv[:, None]
    dinv_ref[...] = dinv

  return pl.pallas_call(
      body,
      grid=(_NBLK,),
      in_specs=[
          pl.BlockSpec((_BLK,), lambda i: (i,)),
          pl.BlockSpec((_BLK,), lambda i: (_NBLK + i,)),
          pl.BlockSpec((_BLK, 128), lambda i: (i, 0)),
          pl.BlockSpec((128, 16), lambda i: (0, 0)),
      ],
      out_specs=[
          pl.BlockSpec((_BLK, 16), lambda i: (i, 0)),
          pl.BlockSpec((_BLK,), lambda i: (i,)),
      ],
      out_shape=[
          jax.ShapeDtypeStruct((_N_PAD, 16), jnp.float32),
          jax.ShapeDtypeStruct((_N_PAD,), jnp.float32),
      ],
  )(pdeg_flat, pdeg_flat, x_pad, W1)


def _tc_layer2(p1, g1, dinv, b1, W2):
  def body(p0_ref, p1_ref, g1_ref, dinv_ref, b1_ref, w_ref, g2_ref):
    dinv = dinv_ref[...]
    s = p0_ref[...] + p1_ref[...] + g1_ref[...]
    out1 = jnp.maximum(s * dinv[:, None] + b1_ref[...], 0.0)
    h2 = jnp.dot(out1, w_ref[...], preferred_element_type=jnp.float32)
    g2_ref[...] = h2 * dinv[:, None]

  return pl.pallas_call(
      body,
      grid=(_NBLK,),
      in_specs=[
          pl.BlockSpec((_BLK, 16), lambda i: (i, 0)),
          pl.BlockSpec((_BLK, 16), lambda i: (_NBLK + i, 0)),
          pl.BlockSpec((_BLK, 16), lambda i: (i, 0)),
          pl.BlockSpec((_BLK,), lambda i: (i,)),
          pl.BlockSpec((1, 16), lambda i: (0, 0)),
          pl.BlockSpec((16, 32), lambda i: (0, 0)),
      ],
      out_specs=pl.BlockSpec((_BLK, 32), lambda i: (i, 0)),
      out_shape=jax.ShapeDtypeStruct((_N_PAD, 32), jnp.float32),
  )(p1, p1, g1, dinv, b1, W2)


def _tc_final(p2, g2, dinv, b2, batch_pad, Wfc, bfc):
  nc = Wfc.shape[1]

  def body(q0_ref, q1_ref, g2_ref, dinv_ref, b2_ref, batch_ref, wfc_ref,
           bfc_ref, out_ref, acc, cnt):
    i = pl.program_id(0)
    dinv = dinv_ref[...]
    s = q0_ref[...] + q1_ref[...] + g2_ref[...]
    out2 = jnp.maximum(s * dinv[:, None] + b2_ref[...], 0.0)
    seg = batch_ref[...]
    gids = lax.broadcasted_iota(jnp.int32, (_BLK, _G), 1)
    mask = (seg[:, None] == gids).astype(jnp.float32)
    pooled_p = lax.dot_general(mask, out2, (((0,), (0,)), ((), ())))
    ones = jnp.ones((_BLK, 1), jnp.float32)
    cnt_p = lax.dot_general(mask, ones, (((0,), (0,)), ((), ())))

    @pl.when(i == 0)
    def _():
      acc[...] = pooled_p
      cnt[...] = cnt_p

    @pl.when(i > 0)
    def _():
      acc[...] += pooled_p
      cnt[...] += cnt_p

    @pl.when(i == _NBLK - 1)
    def _():
      pooled = acc[...] / jnp.maximum(cnt[...], 1.0)
      logits = jnp.dot(pooled, wfc_ref[...],
                       preferred_element_type=jnp.float32) + bfc_ref[...]
      m = jnp.max(logits, axis=1, keepdims=True)
      lse = m + jnp.log(jnp.sum(jnp.exp(logits - m), axis=1, keepdims=True))
      out_ref[...] = logits - lse

  return pl.pallas_call(
      body,
      grid=(_NBLK,),
      in_specs=[
          pl.BlockSpec((_BLK, 32), lambda i: (i, 0)),
          pl.BlockSpec((_BLK, 32), lambda i: (_NBLK + i, 0)),
          pl.BlockSpec((_BLK, 32), lambda i: (i, 0)),
          pl.BlockSpec((_BLK,), lambda i: (i,)),
          pl.BlockSpec((1, 32), lambda i: (0, 0)),
          pl.BlockSpec((_BLK,), lambda i: (i,)),
          pl.BlockSpec((32, nc), lambda i: (0, 0)),
          pl.BlockSpec((1, nc), lambda i: (0, 0)),
      ],
      out_specs=pl.BlockSpec((_G, nc), lambda i: (0, 0)),
      out_shape=jax.ShapeDtypeStruct((_G, nc), jnp.float32),
      scratch_shapes=[
          pltpu.VMEM((_G, 32), jnp.float32),
          pltpu.VMEM((_G, 1), jnp.float32),
      ],
  )(p2, p2, g2, dinv, b2, batch_pad, Wfc, bfc)


def kernel(x, edge_index, batch, W1, b1, W2, b2, Wfc, bfc):
  src = edge_index[0]
  dst = edge_index[1]
  epad = _E_PAD - _E
  pad_idx = _N + (jnp.arange(epad, dtype=jnp.int32) % (_N_PAD - _N))
  src_pad = jnp.concatenate([src, pad_idx])
  dst_pad = jnp.concatenate([dst, pad_idx])
  x_pad = jnp.pad(x, ((0, _N_PAD - _N), (0, 0)))
  batch_pad = jnp.concatenate(
      [batch, jnp.full((_N_PAD - _N,), _G, jnp.int32)])

  pdeg = _sc_degree(dst_pad)
  g1, dinv = _tc_layer1(pdeg, x_pad, W1)
  p1 = _sc_aggregate(g1, src_pad, dst_pad, 16)
  g2 = _tc_layer2(p1, g1, dinv, b1.reshape(1, -1), W2)
  p2 = _sc_aggregate(g2, src_pad, dst_pad, 32)
  return _tc_final(p2, g2, dinv, b2.reshape(1, -1), batch_pad,
                   Wfc, bfc.reshape(1, -1))


# speedup vs baseline: 57.2588x; 2.6296x over previous
import functools

import jax
import jax.numpy as jnp
from jax import lax
from jax.experimental import pallas as pl
from jax.experimental.pallas import tpu as pltpu
from jax.experimental.pallas import tpu_sc as plsc

_N = 10000
_E = 320000
_G = 128

_NCORES = 2
_NSUB = 16
_NW = _NCORES * _NSUB
_CHUNK = 128

_N_PAD = 10240
_ROWS_PER_TILE = _N_PAD // _NSUB
_EPW = ((_E + _NW * _CHUNK - 1) // (_NW * _CHUNK)) * _CHUNK
_E_PAD = _EPW * _NW
_CHUNKS_PER_W = _EPW // _CHUNK

_BLK = 1024
_NBLK = _N_PAD // _BLK


_SC_PARAMS = pltpu.CompilerParams(use_tc_tiling_on_sc=False)


def _sc_degree(dst3):
  mesh = plsc.VectorSubcoreMesh(core_axis_name="c", subcore_axis_name="s")
  depth = 8

  @functools.partial(
      pl.kernel,
      out_type=jax.ShapeDtypeStruct((_NCORES * _N_PAD,), jnp.float32),
      mesh=mesh,
      compiler_params=_SC_PARAMS,
      scratch_types=[
          pltpu.VMEM((_CHUNKS_PER_W, _CHUNK), jnp.int32),
          pltpu.VMEM((_CHUNK,), jnp.float32),
          pltpu.VMEM((_ROWS_PER_TILE,), jnp.float32),
          pltpu.VMEM_SHARED((_N_PAD,), jnp.float32),
          pltpu.SemaphoreType.DMA,
      ],
  )
  def k(dst_hbm, out_hbm, idx_v, ones_v, zbuf_v, acc_sh, sem):
    c = lax.axis_index("c")
    s = lax.axis_index("s")
    w = c * _NSUB + s
    for j in range(_CHUNK // 16):
      ones_v[pl.ds(j * 16, 16)] = jnp.ones((16,), jnp.float32)

    def zfill(i, carry):
      zbuf_v[pl.ds(i * 16, 16)] = jnp.zeros((16,), jnp.float32)
      return carry

    lax.fori_loop(0, _ROWS_PER_TILE // 16, zfill, 0)
    pltpu.sync_copy(zbuf_v, acc_sh.at[pl.ds(s * _ROWS_PER_TILE, _ROWS_PER_TILE)])
    pltpu.sync_copy(dst_hbm.at[w], idx_v)
    plsc.subcore_barrier()

    pend = [None] * _CHUNKS_PER_W
    for i in range(_CHUNKS_PER_W):
      if i >= depth:
        pend[i - depth].wait()
      pend[i] = pltpu.async_copy(ones_v, acc_sh.at[idx_v.at[i]], sem, add=True)
    for i in range(_CHUNKS_PER_W - depth, _CHUNKS_PER_W):
      pend[i].wait()

    plsc.subcore_barrier()
    pltpu.sync_copy(
        acc_sh.at[pl.ds(s * _ROWS_PER_TILE, _ROWS_PER_TILE)],
        out_hbm.at[pl.ds(c * _N_PAD + s * _ROWS_PER_TILE, _ROWS_PER_TILE)])

  return k(dst3)


def _sc_aggregate(g, src3, dst3, d):
  mesh = plsc.VectorSubcoreMesh(core_axis_name="c", subcore_axis_name="s")
  nb = 4
  lag = 2

  @functools.partial(
      pl.kernel,
      out_type=jax.ShapeDtypeStruct((_NCORES * _N_PAD, d), jnp.float32),
      mesh=mesh,
      compiler_params=_SC_PARAMS,
      scratch_types=[
          pltpu.VMEM((_CHUNKS_PER_W, _CHUNK), jnp.int32),
          pltpu.VMEM((_CHUNKS_PER_W, _CHUNK), jnp.int32),
          [pltpu.VMEM((_CHUNK, d), jnp.float32) for _ in range(nb)],
          pltpu.VMEM((_ROWS_PER_TILE, d), jnp.float32),
          pltpu.VMEM_SHARED((_N_PAD, d), jnp.float32),
          [pltpu.SemaphoreType.DMA for _ in range(nb)],
          [pltpu.SemaphoreType.DMA for _ in range(nb)],
      ],
  )
  def k(g_hbm, src_hbm, dst_hbm, out_hbm, src_v, dst_v, rows_v, zbuf_v,
        acc_sh, gsem, ssem):
    c = lax.axis_index("c")
    s = lax.axis_index("s")
    w = c * _NSUB + s

    def zfill(i, carry):
      for j in range(d // 16):
        zbuf_v[i, pl.ds(j * 16, 16)] = jnp.zeros((16,), jnp.float32)
      return carry

    lax.fori_loop(0, _ROWS_PER_TILE, zfill, 0)
    pltpu.sync_copy(zbuf_v, acc_sh.at[pl.ds(s * _ROWS_PER_TILE, _ROWS_PER_TILE)])
    pltpu.sync_copy(src_hbm.at[w], src_v)
    pltpu.sync_copy(dst_hbm.at[w], dst_v)
    plsc.subcore_barrier()

    gd = [None] * _CHUNKS_PER_W
    sd = [None] * _CHUNKS_PER_W
    for t in range(_CHUNKS_PER_W + lag):
      if t < _CHUNKS_PER_W:
        b = t % nb
        if t >= nb:
          sd[t - nb].wait()
        gd[t] = pltpu.async_copy(g_hbm.at[src_v.at[t]], rows_v[b], gsem[b])
      j = t - lag
      if 0 <= j < _CHUNKS_PER_W:
        gd[j].wait()
        sd[j] = pltpu.async_copy(rows_v[j % nb], acc_sh.at[dst_v.at[j]],
                                 ssem[j % nb], add=True)
    for j in range(_CHUNKS_PER_W - nb, _CHUNKS_PER_W):
      sd[j].wait()

    plsc.subcore_barrier()
    pltpu.sync_copy(
        acc_sh.at[pl.ds(s * _ROWS_PER_TILE, _ROWS_PER_TILE)],
        out_hbm.at[pl.ds(c * _N_PAD + s * _ROWS_PER_TILE, _ROWS_PER_TILE)])

  return k(g, src3, dst3)


def _tc_layer1(pdeg_flat, x_pad, W1):
  def body(pd0_ref, pd1_ref, x_ref, w_ref, g_ref, dinv_ref):
    deg = 1.0 + pd0_ref[...] + pd1_ref[...]
    dinv = lax.rsqrt(deg)
    h = jnp.dot(x_ref[...], w_ref[...], preferred_element_type=jnp.float32)
    g_ref[...] = h * dinv[:, None]
    dinv_ref[...] = dinv

  return pl.pallas_call(
      body,
      grid=(_NBLK,),
      in_specs=[
          pl.BlockSpec((_BLK,), lambda i: (i,)),
          pl.BlockSpec((_BLK,), lambda i: (_NBLK + i,)),
          pl.BlockSpec((_BLK, 128), lambda i: (i, 0)),
          pl.BlockSpec((128, 16), lambda i: (0, 0)),
      ],
      out_specs=[
          pl.BlockSpec((_BLK, 16), lambda i: (i, 0)),
          pl.BlockSpec((_BLK,), lambda i: (i,)),
      ],
      out_shape=[
          jax.ShapeDtypeStruct((_N_PAD, 16), jnp.float32),
          jax.ShapeDtypeStruct((_N_PAD,), jnp.float32),
      ],
  )(pdeg_flat, pdeg_flat, x_pad, W1)


def _tc_layer2(p1, g1, dinv, b1, W2):
  def body(p0_ref, p1_ref, g1_ref, dinv_ref, b1_ref, w_ref, g2_ref):
    dinv = dinv_ref[...]
    s = p0_ref[...] + p1_ref[...] + g1_ref[...]
    out1 = jnp.maximum(s * dinv[:, None] + b1_ref[...], 0.0)
    h2 = jnp.dot(out1, w_ref[...], preferred_element_type=jnp.float32)
    g2_ref[...] = h2 * dinv[:, None]

  return pl.pallas_call(
      body,
      grid=(_NBLK,),
      in_specs=[
          pl.BlockSpec((_BLK, 16), lambda i: (i, 0)),
          pl.BlockSpec((_BLK, 16), lambda i: (_NBLK + i, 0)),
          pl.BlockSpec((_BLK, 16), lambda i: (i, 0)),
          pl.BlockSpec((_BLK,), lambda i: (i,)),
          pl.BlockSpec((1, 16), lambda i: (0, 0)),
          pl.BlockSpec((16, 32), lambda i: (0, 0)),
      ],
      out_specs=pl.BlockSpec((_BLK, 32), lambda i: (i, 0)),
      out_shape=jax.ShapeDtypeStruct((_N_PAD, 32), jnp.float32),
  )(p1, p1, g1, dinv, b1, W2)


def _tc_final(p2, g2, dinv, b2, batch_pad, Wfc, bfc):
  nc = Wfc.shape[1]

  def body(q0_ref, q1_ref, g2_ref, dinv_ref, b2_ref, batch_ref, wfc_ref,
           bfc_ref, out_ref, acc, cnt):
    i = pl.program_id(0)
    dinv = dinv_ref[...]
    s = q0_ref[...] + q1_ref[...] + g2_ref[...]
    out2 = jnp.maximum(s * dinv[:, None] + b2_ref[...], 0.0)
    seg = batch_ref[...]
    gids = lax.broadcasted_iota(jnp.int32, (_BLK, _G), 1)
    mask = (seg[:, None] == gids).astype(jnp.float32)
    pooled_p = lax.dot_general(mask, out2, (((0,), (0,)), ((), ())))
    ones = jnp.ones((_BLK, 1), jnp.float32)
    cnt_p = lax.dot_general(mask, ones, (((0,), (0,)), ((), ())))

    @pl.when(i == 0)
    def _():
      acc[...] = pooled_p
      cnt[...] = cnt_p

    @pl.when(i > 0)
    def _():
      acc[...] += pooled_p
      cnt[...] += cnt_p

    @pl.when(i == _NBLK - 1)
    def _():
      pooled = acc[...] / jnp.maximum(cnt[...], 1.0)
      logits = jnp.dot(pooled, wfc_ref[...],
                       preferred_element_type=jnp.float32) + bfc_ref[...]
      m = jnp.max(logits, axis=1, keepdims=True)
      lse = m + jnp.log(jnp.sum(jnp.exp(logits - m), axis=1, keepdims=True))
      out_ref[...] = logits - lse

  return pl.pallas_call(
      body,
      grid=(_NBLK,),
      in_specs=[
          pl.BlockSpec((_BLK, 32), lambda i: (i, 0)),
          pl.BlockSpec((_BLK, 32), lambda i: (_NBLK + i, 0)),
          pl.BlockSpec((_BLK, 32), lambda i: (i, 0)),
          pl.BlockSpec((_BLK,), lambda i: (i,)),
          pl.BlockSpec((1, 32), lambda i: (0, 0)),
          pl.BlockSpec((_BLK,), lambda i: (i,)),
          pl.BlockSpec((32, nc), lambda i: (0, 0)),
          pl.BlockSpec((1, nc), lambda i: (0, 0)),
      ],
      out_specs=pl.BlockSpec((_G, nc), lambda i: (0, 0)),
      out_shape=jax.ShapeDtypeStruct((_G, nc), jnp.float32),
      scratch_shapes=[
          pltpu.VMEM((_G, 32), jnp.float32),
          pltpu.VMEM((_G, 1), jnp.float32),
      ],
  )(p2, p2, g2, dinv, b2, batch_pad, Wfc, bfc)


def kernel(x, edge_index, batch, W1, b1, W2, b2, Wfc, bfc):
  src = edge_index[0]
  dst = edge_index[1]
  epad = _E_PAD - _E
  pad_idx = _N + (jnp.arange(epad, dtype=jnp.int32) % (_N_PAD - _N))
  src3 = jnp.concatenate([src, pad_idx]).reshape(_NW, _CHUNKS_PER_W, _CHUNK)
  dst3 = jnp.concatenate([dst, pad_idx]).reshape(_NW, _CHUNKS_PER_W, _CHUNK)
  x_pad = jnp.pad(x, ((0, _N_PAD - _N), (0, 0)))
  batch_pad = jnp.concatenate(
      [batch, jnp.full((_N_PAD - _N,), _G, jnp.int32)])

  pdeg = _sc_degree(dst3)
  g1, dinv = _tc_layer1(pdeg, x_pad, W1)
  p1 = _sc_aggregate(g1, src3, dst3, 16)
  g2 = _tc_layer2(p1, g1, dinv, b1.reshape(1, -1), W2)
  p2 = _sc_aggregate(g2, src3, dst3, 32)
  return _tc_final(p2, g2, dinv, b2.reshape(1, -1), batch_pad,
                   Wfc, bfc.reshape(1, -1))


# speedup vs baseline: 61.2679x; 1.0700x over previous
import functools

import jax
import jax.numpy as jnp
from jax import lax
from jax.experimental import pallas as pl
from jax.experimental.pallas import tpu as pltpu
from jax.experimental.pallas import tpu_sc as plsc

_N = 10000
_E = 320000
_G = 128

_NCORES = 2
_NSUB = 16
_NW = _NCORES * _NSUB
_CHUNK = 128
_NCHUNKS = _E // _CHUNK
_CPW = _NCHUNKS // _NW
_NTAIL = _NCHUNKS - _CPW * _NW

_N_PAD = 10240
_ROWS_PER_TILE = _N_PAD // _NSUB

_BLK = 1024
_NBLK = (_N + _BLK - 1) // _BLK

_SC_PARAMS = pltpu.CompilerParams(use_tc_tiling_on_sc=False)


def _sc_degree(ei3):
  mesh = plsc.VectorSubcoreMesh(core_axis_name="c", subcore_axis_name="s")
  depth = 8

  @functools.partial(
      pl.kernel,
      out_type=[jax.ShapeDtypeStruct((_N_PAD,), jnp.float32),
                jax.ShapeDtypeStruct((_N_PAD,), jnp.float32)],
      mesh=mesh,
      compiler_params=_SC_PARAMS,
      scratch_types=[
          pltpu.VMEM((_CPW + 1, _CHUNK), jnp.int32),
          pltpu.VMEM((_CHUNK,), jnp.float32),
          pltpu.VMEM((_ROWS_PER_TILE,), jnp.float32),
          pltpu.VMEM_SHARED((_N_PAD,), jnp.float32),
          pltpu.SemaphoreType.DMA,
          pltpu.SemaphoreType.DMA,
      ],
  )
  def k(ei_hbm, out0_hbm, out1_hbm, idx_v, ones_v, zbuf_v, acc_sh, isem, sem):
    c = lax.axis_index("c")
    s = lax.axis_index("s")
    w = c * _NSUB + s
    for j in range(_CHUNK // 16):
      ones_v[pl.ds(j * 16, 16)] = jnp.ones((16,), jnp.float32)

    ipend = [pltpu.async_copy(ei_hbm.at[1, w * _CPW + i], idx_v.at[i], isem)
             for i in range(_CPW)]

    def zfill(i, carry):
      zbuf_v[pl.ds(i * 16, 16)] = jnp.zeros((16,), jnp.float32)
      return carry

    lax.fori_loop(0, _ROWS_PER_TILE // 16, zfill, 0)
    pltpu.sync_copy(zbuf_v, acc_sh.at[pl.ds(s * _ROWS_PER_TILE, _ROWS_PER_TILE)])

    @pl.when(w < _NTAIL)
    def _():
      pltpu.sync_copy(ei_hbm.at[1, _CPW * _NW + w], idx_v.at[_CPW])

    for p in ipend:
      p.wait()
    plsc.subcore_barrier()

    pend = [None] * _CPW
    for i in range(_CPW):
      if i >= depth:
        pend[i - depth].wait()
      pend[i] = pltpu.async_copy(ones_v, acc_sh.at[idx_v.at[i]], sem, add=True)
    for i in range(_CPW - depth, _CPW):
      pend[i].wait()

    @pl.when(w < _NTAIL)
    def _():
      pltpu.sync_copy(ones_v, acc_sh.at[idx_v.at[_CPW]], add=True)

    plsc.subcore_barrier()
    row0 = pl.ds(s * _ROWS_PER_TILE, _ROWS_PER_TILE)

    @pl.when(c == 0)
    def _():
      pltpu.sync_copy(acc_sh.at[row0], out0_hbm.at[row0])

    @pl.when(c == 1)
    def _():
      pltpu.sync_copy(acc_sh.at[row0], out1_hbm.at[row0])

  return k(ei3)


def _sc_aggregate(g, ei3, d):
  mesh = plsc.VectorSubcoreMesh(core_axis_name="c", subcore_axis_name="s")
  nb = 4
  lag = 2

  @functools.partial(
      pl.kernel,
      out_type=[jax.ShapeDtypeStruct((_N_PAD, d), jnp.float32),
                jax.ShapeDtypeStruct((_N_PAD, d), jnp.float32)],
      mesh=mesh,
      compiler_params=_SC_PARAMS,
      scratch_types=[
          pltpu.VMEM((_CPW + 1, _CHUNK), jnp.int32),
          pltpu.VMEM((_CPW + 1, _CHUNK), jnp.int32),
          [pltpu.VMEM((_CHUNK, d), jnp.float32) for _ in range(nb)],
          pltpu.VMEM((_ROWS_PER_TILE, d), jnp.float32),
          pltpu.VMEM_SHARED((_N_PAD, d), jnp.float32),
          pltpu.SemaphoreType.DMA,
          [pltpu.SemaphoreType.DMA for _ in range(nb)],
          [pltpu.SemaphoreType.DMA for _ in range(nb)],
      ],
  )
  def k(g_hbm, ei_hbm, out0_hbm, out1_hbm, src_v, dst_v, rows_v, zbuf_v,
        acc_sh, isem, gsem, ssem):
    c = lax.axis_index("c")
    s = lax.axis_index("s")
    w = c * _NSUB + s

    ipend = []
    for i in range(_CPW):
      ipend.append(
          pltpu.async_copy(ei_hbm.at[0, w * _CPW + i], src_v.at[i], isem))
      ipend.append(
          pltpu.async_copy(ei_hbm.at[1, w * _CPW + i], dst_v.at[i], isem))

    def zfill(i, carry):
      for j in range(d // 16):
        zbuf_v[i, pl.ds(j * 16, 16)] = jnp.zeros((16,), jnp.float32)
      return carry

    lax.fori_loop(0, _ROWS_PER_TILE, zfill, 0)
    pltpu.sync_copy(zbuf_v, acc_sh.at[pl.ds(s * _ROWS_PER_TILE, _ROWS_PER_TILE)])

    @pl.when(w < _NTAIL)
    def _():
      pltpu.sync_copy(ei_hbm.at[0, _CPW * _NW + w], src_v.at[_CPW])
      pltpu.sync_copy(ei_hbm.at[1, _CPW * _NW + w], dst_v.at[_CPW])

    for p in ipend:
      p.wait()
    plsc.subcore_barrier()

    gd = [None] * _CPW
    sd = [None] * _CPW
    for t in range(_CPW + lag):
      if t < _CPW:
        b = t % nb
        if t >= nb:
          sd[t - nb].wait()
        gd[t] = pltpu.async_copy(g_hbm.at[src_v.at[t]], rows_v[b], gsem[b])
      j = t - lag
      if 0 <= j < _CPW:
        gd[j].wait()
        sd[j] = pltpu.async_copy(rows_v[j % nb], acc_sh.at[dst_v.at[j]],
                                 ssem[j % nb], add=True)
    for j in range(_CPW - nb, _CPW):
      sd[j].wait()

    @pl.when(w < _NTAIL)
    def _():
      pltpu.async_copy(g_hbm.at[src_v.at[_CPW]], rows_v[0], gsem[0]).wait()
      pltpu.sync_copy(rows_v[0], acc_sh.at[dst_v.at[_CPW]], add=True)

    plsc.subcore_barrier()
    row0 = pl.ds(s * _ROWS_PER_TILE, _ROWS_PER_TILE)

    @pl.when(c == 0)
    def _():
      pltpu.sync_copy(acc_sh.at[row0], out0_hbm.at[row0])

    @pl.when(c == 1)
    def _():
      pltpu.sync_copy(acc_sh.at[row0], out1_hbm.at[row0])

  return k(g, ei3)


def _tc_layer1(pd0, pd1, x, W1):
  def body(pd0_ref, pd1_ref, x_ref, w_ref, g_ref, dinv_ref):
    deg = 1.0 + pd0_ref[...] + pd1_ref[...]
    dinv = lax.rsqrt(deg)
    h = jnp.dot(x_ref[...], w_ref[...], preferred_element_type=jnp.float32)
    g_ref[...] = h * dinv[:, None]
    dinv_ref[...] = dinv

  return pl.pallas_call(
      body,
      grid=(_NBLK,),
      in_specs=[
          pl.BlockSpec((_BLK,), lambda i: (i,)),
          pl.BlockSpec((_BLK,), lambda i: (i,)),
          pl.BlockSpec((_BLK, 128), lambda i: (i, 0)),
          pl.BlockSpec((128, 16), lambda i: (0, 0)),
      ],
      out_specs=[
          pl.BlockSpec((_BLK, 16), lambda i: (i, 0)),
          pl.BlockSpec((_BLK,), lambda i: (i,)),
      ],
      out_shape=[
          jax.ShapeDtypeStruct((_N, 16), jnp.float32),
          jax.ShapeDtypeStruct((_N,), jnp.float32),
      ],
  )(pd0, pd1, x, W1)


def _tc_layer2(p0, p1, g1, dinv, b1, W2):
  def body(p0_ref, p1_ref, g1_ref, dinv_ref, b1_ref, w_ref, g2_ref):
    dinv = dinv_ref[...]
    su = p0_ref[...] + p1_ref[...] + g1_ref[...]
    out1 = jnp.maximum(su * dinv[:, None] + b1_ref[...], 0.0)
    h2 = jnp.dot(out1, w_ref[...], preferred_element_type=jnp.float32)
    g2_ref[...] = h2 * dinv[:, None]

  return pl.pallas_call(
      body,
      grid=(_NBLK,),
      in_specs=[
          pl.BlockSpec((_BLK, 16), lambda i: (i, 0)),
          pl.BlockSpec((_BLK, 16), lambda i: (i, 0)),
          pl.BlockSpec((_BLK, 16), lambda i: (i, 0)),
          pl.BlockSpec((_BLK,), lambda i: (i,)),
          pl.BlockSpec((1, 16), lambda i: (0, 0)),
          pl.BlockSpec((16, 32), lambda i: (0, 0)),
      ],
      out_specs=pl.BlockSpec((_BLK, 32), lambda i: (i, 0)),
      out_shape=jax.ShapeDtypeStruct((_N, 32), jnp.float32),
  )(p0, p1, g1, dinv, b1, W2)


def _tc_final(q0, q1, g2, dinv, b2, batch, Wfc, bfc):
  nc = Wfc.shape[1]

  def body(q0_ref, q1_ref, g2_ref, dinv_ref, b2_ref, batch_ref, wfc_ref,
           bfc_ref, out_ref, acc, cnt):
    i = pl.program_id(0)
    dinv = dinv_ref[...]
    su = q0_ref[...] + q1_ref[...] + g2_ref[...]
    out2 = jnp.maximum(su * dinv[:, None] + b2_ref[...], 0.0)
    seg = batch_ref[...]
    gids = lax.broadcasted_iota(jnp.int32, (_BLK, _G), 1)
    rows = i * _BLK + lax.broadcasted_iota(jnp.int32, (_BLK, _G), 0)
    mask = ((seg[:, None] == gids) & (rows < _N)).astype(jnp.float32)
    out2 = jnp.where(rows[:, :1] < _N, out2, 0.0)
    pooled_p = lax.dot_general(mask, out2, (((0,), (0,)), ((), ())))
    ones = jnp.ones((_BLK, 1), jnp.float32)
    cnt_p = lax.dot_general(mask, ones, (((0,), (0,)), ((), ())))

    @pl.when(i == 0)
    def _():
      acc[...] = pooled_p
      cnt[...] = cnt_p

    @pl.when(i > 0)
    def _():
      acc[...] += pooled_p
      cnt[...] += cnt_p

    @pl.when(i == _NBLK - 1)
    def _():
      pooled = acc[...] / jnp.maximum(cnt[...], 1.0)
      logits = jnp.dot(pooled, wfc_ref[...],
                       preferred_element_type=jnp.float32) + bfc_ref[...]
      m = jnp.max(logits, axis=1, keepdims=True)
      lse = m + jnp.log(jnp.sum(jnp.exp(logits - m), axis=1, keepdims=True))
      out_ref[...] = logits - lse

  return pl.pallas_call(
      body,
      grid=(_NBLK,),
      in_specs=[
          pl.BlockSpec((_BLK, 32), lambda i: (i, 0)),
          pl.BlockSpec((_BLK, 32), lambda i: (i, 0)),
          pl.BlockSpec((_BLK, 32), lambda i: (i, 0)),
          pl.BlockSpec((_BLK,), lambda i: (i,)),
          pl.BlockSpec((1, 32), lambda i: (0, 0)),
          pl.BlockSpec((_BLK,), lambda i: (i,)),
          pl.BlockSpec((32, nc), lambda i: (0, 0)),
          pl.BlockSpec((1, nc), lambda i: (0, 0)),
      ],
      out_specs=pl.BlockSpec((_G, nc), lambda i: (0, 0)),
      out_shape=jax.ShapeDtypeStruct((_G, nc), jnp.float32),
      scratch_shapes=[
          pltpu.VMEM((_G, 32), jnp.float32),
          pltpu.VMEM((_G, 1), jnp.float32),
      ],
  )(q0, q1, g2, dinv, b2, batch, Wfc, bfc)


def kernel(x, edge_index, batch, W1, b1, W2, b2, Wfc, bfc):
  ei3 = edge_index.reshape(2, _NCHUNKS, _CHUNK)

  pd0, pd1 = _sc_degree(ei3)
  g1, dinv = _tc_layer1(pd0, pd1, x, W1)
  p0, p1 = _sc_aggregate(g1, ei3, 16)
  g2 = _tc_layer2(p0, p1, g1, dinv, b1.reshape(1, -1), W2)
  q0, q1 = _sc_aggregate(g2, ei3, 32)
  return _tc_final(q0, q1, g2, dinv, b2.reshape(1, -1), batch,
                   Wfc, bfc.reshape(1, -1))


# speedup vs baseline: 70.4985x; 1.1507x over previous
import functools

import jax
import jax.numpy as jnp
from jax import lax
from jax.experimental import pallas as pl
from jax.experimental.pallas import tpu as pltpu
from jax.experimental.pallas import tpu_sc as plsc

_N = 10000
_E = 320000
_G = 128

_NCORES = 2
_NSUB = 16
_NW = _NCORES * _NSUB
_CHUNK = 128
_NCHUNKS = _E // _CHUNK
_CPW = _NCHUNKS // _NW
_NTAIL = _NCHUNKS - _CPW * _NW

_N_PAD = 10240
_ROWS_PER_TILE = _N_PAD // _NSUB

_BLK = 1024
_NBLK = (_N + _BLK - 1) // _BLK

_SC_PARAMS = pltpu.CompilerParams(use_tc_tiling_on_sc=False)


def _sc_degree(ei3):
  mesh = plsc.VectorSubcoreMesh(core_axis_name="c", subcore_axis_name="s")
  depth = 8

  @functools.partial(
      pl.kernel,
      out_type=[jax.ShapeDtypeStruct((_N_PAD,), jnp.float32),
                jax.ShapeDtypeStruct((_N_PAD,), jnp.float32)],
      mesh=mesh,
      compiler_params=_SC_PARAMS,
      scratch_types=[
          pltpu.VMEM((_CPW + 1, _CHUNK), jnp.int32),
          pltpu.VMEM((_CHUNK,), jnp.float32),
          pltpu.VMEM((_ROWS_PER_TILE,), jnp.float32),
          pltpu.VMEM_SHARED((_N_PAD,), jnp.float32),
          pltpu.SemaphoreType.DMA,
          pltpu.SemaphoreType.DMA,
      ],
  )
  def k(ei_hbm, out0_hbm, out1_hbm, idx_v, ones_v, zbuf_v, acc_sh, isem, sem):
    c = lax.axis_index("c")
    s = lax.axis_index("s")
    w = c * _NSUB + s
    for j in range(_CHUNK // 16):
      ones_v[pl.ds(j * 16, 16)] = jnp.ones((16,), jnp.float32)

    ipend = [pltpu.async_copy(ei_hbm.at[1, w * _CPW + i], idx_v.at[i], isem)
             for i in range(_CPW)]

    def zfill(i, carry):
      zbuf_v[pl.ds(i * 16, 16)] = jnp.zeros((16,), jnp.float32)
      return carry

    lax.fori_loop(0, _ROWS_PER_TILE // 16, zfill, 0)
    pltpu.sync_copy(zbuf_v, acc_sh.at[pl.ds(s * _ROWS_PER_TILE, _ROWS_PER_TILE)])

    @pl.when(w < _NTAIL)
    def _():
      pltpu.sync_copy(ei_hbm.at[1, _CPW * _NW + w], idx_v.at[_CPW])

    for p in ipend:
      p.wait()
    plsc.subcore_barrier()

    pend = [None] * _CPW
    for i in range(_CPW):
      if i >= depth:
        pend[i - depth].wait()
      pend[i] = pltpu.async_copy(ones_v, acc_sh.at[idx_v.at[i]], sem, add=True)
    for i in range(_CPW - depth, _CPW):
      pend[i].wait()

    @pl.when(w < _NTAIL)
    def _():
      pltpu.sync_copy(ones_v, acc_sh.at[idx_v.at[_CPW]], add=True)

    plsc.subcore_barrier()
    row0 = pl.ds(s * _ROWS_PER_TILE, _ROWS_PER_TILE)

    @pl.when(c == 0)
    def _():
      pltpu.sync_copy(acc_sh.at[row0], out0_hbm.at[row0])

    @pl.when(c == 1)
    def _():
      pltpu.sync_copy(acc_sh.at[row0], out1_hbm.at[row0])

  return k(ei3)


def _sc_aggregate(g, ei3, d):
  mesh = plsc.VectorSubcoreMesh(core_axis_name="c", subcore_axis_name="s")
  nb = 6
  lag = 3
  tslice = _N // _NSUB

  @functools.partial(
      pl.kernel,
      out_type=[jax.ShapeDtypeStruct((_N_PAD, d), jnp.float32),
                jax.ShapeDtypeStruct((_N_PAD, d), jnp.float32)],
      mesh=mesh,
      compiler_params=_SC_PARAMS,
      scratch_types=[
          pltpu.VMEM((_CPW + 1, _CHUNK), jnp.int32),
          pltpu.VMEM((_CPW + 1, _CHUNK), jnp.int32),
          [pltpu.VMEM((_CHUNK, d), jnp.float32) for _ in range(nb)],
          pltpu.VMEM((_ROWS_PER_TILE, d), jnp.float32),
          pltpu.VMEM_SHARED((_N_PAD, d), jnp.float32),
          pltpu.VMEM_SHARED((_N, d), jnp.float32),
          pltpu.SemaphoreType.DMA,
          [pltpu.SemaphoreType.DMA for _ in range(nb)],
          [pltpu.SemaphoreType.DMA for _ in range(nb)],
      ],
  )
  def k(g_hbm, ei_hbm, out0_hbm, out1_hbm, src_v, dst_v, rows_v, zbuf_v,
        acc_sh, tbl_sh, isem, gsem, ssem):
    c = lax.axis_index("c")
    s = lax.axis_index("s")
    w = c * _NSUB + s

    tpend = pltpu.async_copy(g_hbm.at[pl.ds(s * tslice, tslice)],
                             tbl_sh.at[pl.ds(s * tslice, tslice)], isem)

    ipend = []
    for i in range(_CPW):
      ipend.append(
          pltpu.async_copy(ei_hbm.at[0, w * _CPW + i], src_v.at[i], isem))
      ipend.append(
          pltpu.async_copy(ei_hbm.at[1, w * _CPW + i], dst_v.at[i], isem))

    def zfill(i, carry):
      for j in range(d // 16):
        zbuf_v[i, pl.ds(j * 16, 16)] = jnp.zeros((16,), jnp.float32)
      return carry

    lax.fori_loop(0, _ROWS_PER_TILE, zfill, 0)
    pltpu.sync_copy(zbuf_v, acc_sh.at[pl.ds(s * _ROWS_PER_TILE, _ROWS_PER_TILE)])

    @pl.when(w < _NTAIL)
    def _():
      pltpu.sync_copy(ei_hbm.at[0, _CPW * _NW + w], src_v.at[_CPW])
      pltpu.sync_copy(ei_hbm.at[1, _CPW * _NW + w], dst_v.at[_CPW])

    tpend.wait()
    for p in ipend:
      p.wait()
    plsc.subcore_barrier()

    gd = [None] * _CPW
    sd = [None] * _CPW
    for t in range(_CPW + lag):
      if t < _CPW:
        b = t % nb
        if t >= nb:
          sd[t - nb].wait()
        gd[t] = pltpu.async_copy(tbl_sh.at[src_v.at[t]], rows_v[b], gsem[b])
      j = t - lag
      if 0 <= j < _CPW:
        gd[j].wait()
        sd[j] = pltpu.async_copy(rows_v[j % nb], acc_sh.at[dst_v.at[j]],
                                 ssem[j % nb], add=True)
    for j in range(_CPW - nb, _CPW):
      sd[j].wait()

    @pl.when(w < _NTAIL)
    def _():
      pltpu.async_copy(tbl_sh.at[src_v.at[_CPW]], rows_v[0], gsem[0]).wait()
      pltpu.sync_copy(rows_v[0], acc_sh.at[dst_v.at[_CPW]], add=True)

    plsc.subcore_barrier()
    row0 = pl.ds(s * _ROWS_PER_TILE, _ROWS_PER_TILE)

    @pl.when(c == 0)
    def _():
      pltpu.sync_copy(acc_sh.at[row0], out0_hbm.at[row0])

    @pl.when(c == 1)
    def _():
      pltpu.sync_copy(acc_sh.at[row0], out1_hbm.at[row0])

  return k(g, ei3)


def _tc_layer1(pd0, pd1, x, W1):
  def body(pd0_ref, pd1_ref, x_ref, w_ref, g_ref, dinv_ref):
    deg = 1.0 + pd0_ref[pl.ds(0, _N)] + pd1_ref[pl.ds(0, _N)]
    dinv = lax.rsqrt(deg)
    h = jnp.dot(x_ref[...], w_ref[...], preferred_element_type=jnp.float32)
    g_ref[...] = h * dinv[:, None]
    dinv_ref[...] = dinv

  return pl.pallas_call(
      body,
      out_shape=[
          jax.ShapeDtypeStruct((_N, 16), jnp.float32),
          jax.ShapeDtypeStruct((_N,), jnp.float32),
      ],
  )(pd0, pd1, x, W1)


def _tc_layer2(p0, p1, g1, dinv, b1, W2):
  def body(p0_ref, p1_ref, g1_ref, dinv_ref, b1_ref, w_ref, g2_ref):
    dinv = dinv_ref[...]
    su = p0_ref[pl.ds(0, _N), :] + p1_ref[pl.ds(0, _N), :] + g1_ref[...]
    out1 = jnp.maximum(su * dinv[:, None] + b1_ref[...], 0.0)
    h2 = jnp.dot(out1, w_ref[...], preferred_element_type=jnp.float32)
    g2_ref[...] = h2 * dinv[:, None]

  return pl.pallas_call(
      body,
      out_shape=jax.ShapeDtypeStruct((_N, 32), jnp.float32),
  )(p0, p1, g1, dinv, b1, W2)


def _tc_final(q0, q1, g2, dinv, b2, batch, Wfc, bfc):
  nc = Wfc.shape[1]

  def body(q0_ref, q1_ref, g2_ref, dinv_ref, b2_ref, batch_ref, wfc_ref,
           bfc_ref, out_ref):
    dinv = dinv_ref[...]
    su = q0_ref[pl.ds(0, _N), :] + q1_ref[pl.ds(0, _N), :] + g2_ref[...]
    out2 = jnp.maximum(su * dinv[:, None] + b2_ref[...], 0.0)
    seg = batch_ref[...]
    gids = lax.broadcasted_iota(jnp.int32, (_N, _G), 1)
    mask = (seg[:, None] == gids).astype(jnp.float32)
    pooled = lax.dot_general(mask, out2, (((0,), (0,)), ((), ())))
    ones = jnp.ones((_N, 1), jnp.float32)
    cnt = lax.dot_general(mask, ones, (((0,), (0,)), ((), ())))
    pooled = pooled / jnp.maximum(cnt, 1.0)
    logits = jnp.dot(pooled, wfc_ref[...],
                     preferred_element_type=jnp.float32) + bfc_ref[...]
    m = jnp.max(logits, axis=1, keepdims=True)
    lse = m + jnp.log(jnp.sum(jnp.exp(logits - m), axis=1, keepdims=True))
    out_ref[...] = logits - lse

  return pl.pallas_call(
      body,
      out_shape=jax.ShapeDtypeStruct((_G, nc), jnp.float32),
  )(q0, q1, g2, dinv, b2, batch, Wfc, bfc)


def kernel(x, edge_index, batch, W1, b1, W2, b2, Wfc, bfc):
  ei3 = edge_index.reshape(2, _NCHUNKS, _CHUNK)

  pd0, pd1 = _sc_degree(ei3)
  g1, dinv = _tc_layer1(pd0, pd1, x, W1)
  p0, p1 = _sc_aggregate(g1, ei3, 16)
  g2 = _tc_layer2(p0, p1, g1, dinv, b1.reshape(1, -1), W2)
  q0, q1 = _sc_aggregate(g2, ei3, 32)
  return _tc_final(q0, q1, g2, dinv, b2.reshape(1, -1), batch,
                   Wfc, bfc.reshape(1, -1))


# speedup vs baseline: 71.7640x; 1.0180x over previous
import functools

import jax
import jax.numpy as jnp
from jax import lax
from jax.experimental import pallas as pl
from jax.experimental.pallas import tpu as pltpu
from jax.experimental.pallas import tpu_sc as plsc

_N = 10000
_E = 320000
_G = 128

_NCORES = 2
_NSUB = 16
_NW = _NCORES * _NSUB
_CHUNK = 128
_NCHUNKS = _E // _CHUNK
_CPW = _NCHUNKS // _NW
_NTAIL = _NCHUNKS - _CPW * _NW

_N_PAD = 10240
_ROWS_PER_TILE = _N_PAD // _NSUB

_BLK = 1024
_NBLK = (_N + _BLK - 1) // _BLK

_SC_PARAMS = pltpu.CompilerParams(use_tc_tiling_on_sc=False)


def _sc_degree(ei3):
  mesh = plsc.VectorSubcoreMesh(core_axis_name="c", subcore_axis_name="s")
  depth = 8

  @functools.partial(
      pl.kernel,
      out_type=[jax.ShapeDtypeStruct((_N_PAD,), jnp.float32),
                jax.ShapeDtypeStruct((_N_PAD,), jnp.float32)],
      mesh=mesh,
      compiler_params=_SC_PARAMS,
      scratch_types=[
          pltpu.VMEM((_CPW + 1, _CHUNK), jnp.int32),
          pltpu.VMEM((_CHUNK,), jnp.float32),
          pltpu.VMEM((_ROWS_PER_TILE,), jnp.float32),
          pltpu.VMEM_SHARED((_N_PAD,), jnp.float32),
          pltpu.SemaphoreType.DMA,
          pltpu.SemaphoreType.DMA,
      ],
  )
  def k(ei_hbm, out0_hbm, out1_hbm, idx_v, ones_v, zbuf_v, acc_sh, isem, sem):
    c = lax.axis_index("c")
    s = lax.axis_index("s")
    w = c * _NSUB + s
    for j in range(_CHUNK // 16):
      ones_v[pl.ds(j * 16, 16)] = jnp.ones((16,), jnp.float32)

    ipend = [pltpu.async_copy(ei_hbm.at[1, w * _CPW + i], idx_v.at[i], isem)
             for i in range(_CPW)]

    def zfill(i, carry):
      zbuf_v[pl.ds(i * 16, 16)] = jnp.zeros((16,), jnp.float32)
      return carry

    lax.fori_loop(0, _ROWS_PER_TILE // 16, zfill, 0)
    pltpu.sync_copy(zbuf_v, acc_sh.at[pl.ds(s * _ROWS_PER_TILE, _ROWS_PER_TILE)])

    @pl.when(w < _NTAIL)
    def _():
      pltpu.sync_copy(ei_hbm.at[1, _CPW * _NW + w], idx_v.at[_CPW])

    for p in ipend:
      p.wait()
    plsc.subcore_barrier()

    pend = [None] * _CPW
    for i in range(_CPW):
      if i >= depth:
        pend[i - depth].wait()
      pend[i] = pltpu.async_copy(ones_v, acc_sh.at[idx_v.at[i]], sem, add=True)
    for i in range(_CPW - depth, _CPW):
      pend[i].wait()

    @pl.when(w < _NTAIL)
    def _():
      pltpu.sync_copy(ones_v, acc_sh.at[idx_v.at[_CPW]], add=True)

    plsc.subcore_barrier()
    row0 = pl.ds(s * _ROWS_PER_TILE, _ROWS_PER_TILE)

    @pl.when(c == 0)
    def _():
      pltpu.sync_copy(acc_sh.at[row0], out0_hbm.at[row0])

    @pl.when(c == 1)
    def _():
      pltpu.sync_copy(acc_sh.at[row0], out1_hbm.at[row0])

  return k(ei3)


def _sc_aggregate(g, ei3, d, spmem_table=True):
  mesh = plsc.VectorSubcoreMesh(core_axis_name="c", subcore_axis_name="s")
  nb = 6
  lag = 3
  tslice = _N // _NSUB

  @functools.partial(
      pl.kernel,
      out_type=[jax.ShapeDtypeStruct((_N_PAD, d), jnp.float32),
                jax.ShapeDtypeStruct((_N_PAD, d), jnp.float32)],
      mesh=mesh,
      compiler_params=_SC_PARAMS,
      scratch_types=[
          pltpu.VMEM((_CPW + 1, _CHUNK), jnp.int32),
          pltpu.VMEM((_CPW + 1, _CHUNK), jnp.int32),
          [pltpu.VMEM((_CHUNK, d), jnp.float32) for _ in range(nb)],
          pltpu.VMEM((_ROWS_PER_TILE, d), jnp.float32),
          pltpu.VMEM_SHARED((_N_PAD, d), jnp.float32),
          pltpu.VMEM_SHARED((_N, d) if spmem_table else (8, d), jnp.float32),
          pltpu.SemaphoreType.DMA,
          [pltpu.SemaphoreType.DMA for _ in range(nb)],
          [pltpu.SemaphoreType.DMA for _ in range(nb)],
      ],
  )
  def k(g_hbm, ei_hbm, out0_hbm, out1_hbm, src_v, dst_v, rows_v, zbuf_v,
        acc_sh, tbl_sh, isem, gsem, ssem):
    c = lax.axis_index("c")
    s = lax.axis_index("s")
    w = c * _NSUB + s

    tbl = tbl_sh if spmem_table else g_hbm
    if spmem_table:
      tpend = pltpu.async_copy(g_hbm.at[pl.ds(s * tslice, tslice)],
                               tbl_sh.at[pl.ds(s * tslice, tslice)], isem)

    ipend = []
    for i in range(_CPW):
      ipend.append(
          pltpu.async_copy(ei_hbm.at[0, w * _CPW + i], src_v.at[i], isem))
      ipend.append(
          pltpu.async_copy(ei_hbm.at[1, w * _CPW + i], dst_v.at[i], isem))

    def zfill(i, carry):
      for j in range(d // 16):
        zbuf_v[i, pl.ds(j * 16, 16)] = jnp.zeros((16,), jnp.float32)
      return carry

    lax.fori_loop(0, _ROWS_PER_TILE, zfill, 0)
    pltpu.sync_copy(zbuf_v, acc_sh.at[pl.ds(s * _ROWS_PER_TILE, _ROWS_PER_TILE)])

    @pl.when(w < _NTAIL)
    def _():
      pltpu.sync_copy(ei_hbm.at[0, _CPW * _NW + w], src_v.at[_CPW])
      pltpu.sync_copy(ei_hbm.at[1, _CPW * _NW + w], dst_v.at[_CPW])

    if spmem_table:
      tpend.wait()
    for p in ipend:
      p.wait()
    plsc.subcore_barrier()

    gd = [None] * _CPW
    sd = [None] * _CPW
    for t in range(_CPW + lag):
      if t < _CPW:
        b = t % nb
        if t >= nb:
          sd[t - nb].wait()
        gd[t] = pltpu.async_copy(tbl.at[src_v.at[t]], rows_v[b], gsem[b])
      j = t - lag
      if 0 <= j < _CPW:
        gd[j].wait()
        sd[j] = pltpu.async_copy(rows_v[j % nb], acc_sh.at[dst_v.at[j]],
                                 ssem[j % nb], add=True)
    for j in range(_CPW - nb, _CPW):
      sd[j].wait()

    @pl.when(w < _NTAIL)
    def _():
      pltpu.async_copy(tbl.at[src_v.at[_CPW]], rows_v[0], gsem[0]).wait()
      pltpu.sync_copy(rows_v[0], acc_sh.at[dst_v.at[_CPW]], add=True)

    plsc.subcore_barrier()
    row0 = pl.ds(s * _ROWS_PER_TILE, _ROWS_PER_TILE)

    @pl.when(c == 0)
    def _():
      pltpu.sync_copy(acc_sh.at[row0], out0_hbm.at[row0])

    @pl.when(c == 1)
    def _():
      pltpu.sync_copy(acc_sh.at[row0], out1_hbm.at[row0])

  return k(g, ei3)


def _tc_layer1(pd0, pd1, x, W1):
  def body(pd0_ref, pd1_ref, x_ref, w_ref, g_ref, dinv_ref):
    deg = 1.0 + pd0_ref[pl.ds(0, _N)] + pd1_ref[pl.ds(0, _N)]
    dinv = lax.rsqrt(deg)
    h = jnp.dot(x_ref[...], w_ref[...], preferred_element_type=jnp.float32)
    g_ref[...] = h * dinv[:, None]
    dinv_ref[...] = dinv

  return pl.pallas_call(
      body,
      out_shape=[
          jax.ShapeDtypeStruct((_N, 16), jnp.float32),
          jax.ShapeDtypeStruct((_N,), jnp.float32),
      ],
  )(pd0, pd1, x, W1)


def _tc_layer2(p0, p1, g1, dinv, b1, W2):
  def body(p0_ref, p1_ref, g1_ref, dinv_ref, b1_ref, w_ref, g2_ref):
    dinv = dinv_ref[...]
    su = p0_ref[pl.ds(0, _N), :] + p1_ref[pl.ds(0, _N), :] + g1_ref[...]
    out1 = jnp.maximum(su * dinv[:, None] + b1_ref[...], 0.0)
    h2 = jnp.dot(out1, w_ref[...], preferred_element_type=jnp.float32)
    g2_ref[...] = h2 * dinv[:, None]

  return pl.pallas_call(
      body,
      out_shape=jax.ShapeDtypeStruct((_N, 32), jnp.float32),
  )(p0, p1, g1, dinv, b1, W2)


def _tc_final(q0, q1, g2, dinv, b2, batch, Wfc, bfc):
  nc = Wfc.shape[1]

  def body(q0_ref, q1_ref, g2_ref, dinv_ref, b2_ref, batch_ref, wfc_ref,
           bfc_ref, out_ref):
    dinv = dinv_ref[...]
    su = q0_ref[pl.ds(0, _N), :] + q1_ref[pl.ds(0, _N), :] + g2_ref[...]
    out2 = jnp.maximum(su * dinv[:, None] + b2_ref[...], 0.0)
    seg = batch_ref[...]
    gids = lax.broadcasted_iota(jnp.int32, (_N, _G), 1)
    mask = (seg[:, None] == gids).astype(jnp.float32)
    pooled = lax.dot_general(mask, out2, (((0,), (0,)), ((), ())))
    ones = jnp.ones((_N, 1), jnp.float32)
    cnt = lax.dot_general(mask, ones, (((0,), (0,)), ((), ())))
    pooled = pooled / jnp.maximum(cnt, 1.0)
    logits = jnp.dot(pooled, wfc_ref[...],
                     preferred_element_type=jnp.float32) + bfc_ref[...]
    m = jnp.max(logits, axis=1, keepdims=True)
    lse = m + jnp.log(jnp.sum(jnp.exp(logits - m), axis=1, keepdims=True))
    out_ref[...] = logits - lse

  return pl.pallas_call(
      body,
      out_shape=jax.ShapeDtypeStruct((_G, nc), jnp.float32),
  )(q0, q1, g2, dinv, b2, batch, Wfc, bfc)


def kernel(x, edge_index, batch, W1, b1, W2, b2, Wfc, bfc):
  ei3 = edge_index.reshape(2, _NCHUNKS, _CHUNK)

  pd0, pd1 = _sc_degree(ei3)
  g1, dinv = _tc_layer1(pd0, pd1, x, W1)
  p0, p1 = _sc_aggregate(g1, ei3, 16)
  g2 = _tc_layer2(p0, p1, g1, dinv, b1.reshape(1, -1), W2)
  q0, q1 = _sc_aggregate(g2, ei3, 32, spmem_table=False)
  return _tc_final(q0, q1, g2, dinv, b2.reshape(1, -1), batch,
                   Wfc, bfc.reshape(1, -1))
